# baseline (speedup 1.0000x reference)
import functools

import jax
import jax.numpy as jnp
from jax import lax
from jax.experimental import pallas as pl
from jax.experimental.pallas import tpu as pltpu
from jax.experimental.pallas import tpu_sc as plsc

NC = 2
NS = 16
NW = NC * NS
CS = 8


@functools.partial(jax.jit, static_argnames=("V", "D", "B", "S", "T"))
def _gather_rows(idx_grouped, table, V, D, B, S, T):
    s_per_w = S // NW
    n_steps = s_per_w // CS
    mesh = plsc.VectorSubcoreMesh(core_axis_name="c", subcore_axis_name="s")

    @functools.partial(
        pl.kernel,
        out_type=jax.ShapeDtypeStruct((S, T, D), jnp.float32),
        mesh=mesh,
        compiler_params=pltpu.CompilerParams(use_tc_tiling_on_sc=False),
        scratch_types=[
            pltpu.VMEM((s_per_w, T), jnp.int32),
            pltpu.VMEM((2, CS, T, D), jnp.float32),
            pltpu.SemaphoreType.DMA,
            pltpu.SemaphoreType.DMA,
            pltpu.SemaphoreType.DMA,
            pltpu.SemaphoreType.DMA,
        ],
    )
    def k(idx_hbm, table_hbm, out_hbm, idx_v, rows_v, g0, g1, w0, w1):
        wid = lax.axis_index("s") * NC + lax.axis_index("c")
        s_base = wid * s_per_w
        pltpu.sync_copy(idx_hbm.at[wid], idx_v)

        gsems = (g0, g1)
        wsems = (w0, w1)

        def gathers(i, b):
            for t in range(CS):
                pltpu.async_copy(
                    table_hbm.at[idx_v.at[i * CS + t]],
                    rows_v.at[b].at[t],
                    gsems[b],
                )

        def wait_gathers(b):
            for t in range(CS):
                pltpu.make_async_copy(
                    table_hbm.at[idx_v.at[0]], rows_v.at[b].at[t], gsems[b]
                ).wait()

        def write(i, b):
            pltpu.async_copy(
                rows_v.at[b], out_hbm.at[pl.ds(s_base + i * CS, CS)], wsems[b]
            )

        def wait_write(b):
            pltpu.make_async_copy(
                rows_v.at[b], out_hbm.at[pl.ds(s_base, CS)], wsems[b]
            ).wait()

        gathers(0, 0)
        gathers(1, 1)

        def body(i, _):
            for b in range(2):
                j = i * 2 + b
                wait_gathers(b)
                write(j, b)
                wait_write(b)

                @pl.when(j + 2 < n_steps)
                def _():
                    gathers(j + 2, b)

            return 0

        lax.fori_loop(0, n_steps // 2, body, 0)

    return k(idx_grouped, table)


def kernel(inputs, table):
    V, D = table.shape
    S, T = inputs.shape
    B = inputs.size
    idx_grouped = inputs.reshape(NW, S // NW, T).astype(jnp.int32)
    return _gather_rows(idx_grouped, table, V, D, B, S, T)
